# baseline (speedup 1.0000x reference)
import functools

import jax
import jax.numpy as jnp
from jax import lax
from jax.experimental import pallas as pl
from jax.experimental.pallas import tpu as pltpu
from jax.experimental.pallas import tpu_sc as plsc

_LANES = 16
_NB = 32
_NBUF = 3


@functools.lru_cache(maxsize=None)
def _build_sc_embed(S, B, V, D):
    info = plsc.get_sparse_core_info()
    NC, NS = info.num_cores, info.num_subcores
    NW = NC * NS
    assert S % NW == 0 and B % _NB == 0 and D % _LANES == 0
    n_pass = S // NW
    n_chunk = B // _NB
    n_loop = (n_chunk - 2) // _NBUF
    assert n_loop * _NBUF + 2 == n_chunk
    n_grp = D // _LANES
    mesh = plsc.VectorSubcoreMesh(core_axis_name="c", subcore_axis_name="s")

    @functools.partial(
        pl.kernel,
        mesh=mesh,
        out_type=jax.ShapeDtypeStruct((B, S, D), jnp.float32),
        scratch_types=(
            [pltpu.VMEM((n_pass * B,), jnp.int32)]
            + [pltpu.VMEM((_NB, D), jnp.float32) for _ in range(_NBUF)]
            + [pltpu.VMEM((D,), jnp.float32)]
            + [pltpu.SemaphoreType.DMA for _ in range(2 * _NBUF)]
        ),
    )
    def k(ids_hbm, tok_hbm, pos_hbm, out_hbm, idx_all, r0, r1, r2,
          pos_v, g0, g1, g2, s0, s1, s2):
        rows = (r0, r1, r2)
        gsem = (g0, g1, g2)
        ssem = (s0, s1, s2)
        wid = lax.axis_index("s") * NC + lax.axis_index("c")

        for p in range(n_pass):
            pltpu.sync_copy(ids_hbm.at[wid + NW * p],
                            idx_all.at[pl.ds(p * B, B)])

        for p in range(n_pass):
            s = wid + NW * p
            pltpu.sync_copy(pos_hbm.at[s], pos_v)

            def ivec(kc, p=p):
                return idx_all.at[pl.ds(p * B + kc * _NB, _NB)]

            def out_slc(kc, s=s):
                return out_hbm.at[pl.ds(kc * _NB, _NB), s]

            def compute(b):
                def col_body(j):
                    c0 = j * _LANES
                    pv = pos_v[pl.ds(c0, _LANES)]
                    for r in range(_NB):
                        rows[b][r, pl.ds(c0, _LANES)] += pv

                plsc.parallel_loop(0, n_grp, unroll=4)(col_body)

            def body(kc, b, wait_prev_scatter, start_next_gather):
                nxt = (b + 1) % _NBUF
                pltpu.make_async_copy(
                    tok_hbm.at[ivec(kc)], rows[b], gsem[b]).wait()
                if wait_prev_scatter:
                    pltpu.make_async_copy(
                        rows[nxt], out_slc(kc - 2), ssem[nxt]).wait()
                if start_next_gather:
                    pltpu.async_copy(
                        tok_hbm.at[ivec(kc + 1)], rows[nxt], gsem[nxt])
                compute(b)
                pltpu.async_copy(rows[b], out_slc(kc), ssem[b])

            pltpu.async_copy(tok_hbm.at[ivec(0)], rows[0], gsem[0])

            def outer(t, carry):
                for b in range(_NBUF):
                    kc = t * _NBUF + b
                    nxt = (b + 1) % _NBUF
                    pltpu.make_async_copy(
                        tok_hbm.at[ivec(kc)], rows[b], gsem[b]).wait()

                    @pl.when(kc >= 2)
                    def _():
                        pltpu.make_async_copy(
                            rows[nxt], out_slc(kc - 2), ssem[nxt]).wait()

                    pltpu.async_copy(
                        tok_hbm.at[ivec(kc + 1)], rows[nxt], gsem[nxt])
                    compute(b)
                    pltpu.async_copy(rows[b], out_slc(kc), ssem[b])
                return carry

            lax.fori_loop(0, n_loop, outer, 0)

            body(n_chunk - 2, (n_chunk - 2) % _NBUF,
                 wait_prev_scatter=True, start_next_gather=True)
            body(n_chunk - 1, (n_chunk - 1) % _NBUF,
                 wait_prev_scatter=True, start_next_gather=False)
            pltpu.make_async_copy(
                rows[(n_chunk - 2) % _NBUF], out_slc(n_chunk - 2),
                ssem[(n_chunk - 2) % _NBUF]).wait()
            pltpu.make_async_copy(
                rows[(n_chunk - 1) % _NBUF], out_slc(n_chunk - 1),
                ssem[(n_chunk - 1) % _NBUF]).wait()

    return k


def kernel(input_ids, token_table, pos_table):
    if input_ids.ndim == 1:
        input_ids = input_ids[None, :]
    B, S = input_ids.shape
    V, D = token_table.shape
    ids_t = input_ids.astype(jnp.int32).T
    return _build_sc_embed(S, B, V, D)(ids_t, token_table, pos_table)
